# Initial kernel scaffold; baseline (speedup 1.0000x reference)
import jax
import jax.numpy as jnp
from jax.experimental import pallas as pl


def kernel(x, W, b, gamma, beta):
    raise NotImplementedError("write your pallas kernel here")


# speedup vs baseline: 9.3413x; 9.3413x over previous
import functools

import jax
import jax.numpy as jnp
from jax import lax
from jax.experimental import pallas as pl
from jax.experimental.pallas import tpu as pltpu
from jax.experimental.pallas import tpu_sc as plsc

_B, _C, _N, _K, _O = 8, 64, 2048, 16, 64
_R = 256
_RC = 256
_CH = 128
_NW = 32


def _knn_body(x_ref, xt_ref, idx_ref):
    b = pl.program_id(0)
    xb = x_ref[0]
    xr = xt_ref[0]
    inner = lax.dot_general(
        xr, xb, (((1,), (0,)), ((), ())),
        preferred_element_type=jnp.float32,
        precision=lax.Precision.DEFAULT,
    )
    sq_all = jnp.sum(xb * xb, axis=0)[None, :]
    sq_r = jnp.sum(xr * xr, axis=1, keepdims=True)
    d = sq_r + sq_all - 2.0 * inner
    iota = lax.broadcasted_iota(jnp.int32, (_R, _N), 1)
    base = b * _N
    for t in range(_K + 1):
        m = jnp.min(d, axis=1, keepdims=True)
        cand = jnp.where(d == m, iota, _N)
        j = jnp.min(cand, axis=1, keepdims=True)
        if t > 0:
            idx_ref[0, t - 1, :] = j[:, 0] + base
        d = jnp.where(iota == j, jnp.inf, d)


def _knn(x, xt):
    nb = _N // _R
    return pl.pallas_call(
        _knn_body,
        grid=(_B, nb),
        in_specs=[
            pl.BlockSpec((1, _C, _N), lambda b, i: (b, 0, 0)),
            pl.BlockSpec((1, _R, _C), lambda b, i: (b, i, 0)),
        ],
        out_specs=pl.BlockSpec((1, _K, _R), lambda b, i: (b, 0, i)),
        out_shape=jax.ShapeDtypeStruct((_B, _K, _N), jnp.int32),
    )(x, xt)


def _gather_body(idx_hbm, table_hbm, out_hbm, idx_v, rows_v, sem):
    wid = lax.axis_index("s") * 2 + lax.axis_index("c")
    rows_per_w = (_B * _K * _N) // _NW
    base = wid * rows_per_w
    nch = rows_per_w // _CH

    def body(i, carry):
        off = base + i * _CH
        pltpu.sync_copy(idx_hbm.at[pl.ds(off, _CH)], idx_v)
        pltpu.async_copy(table_hbm.at[idx_v], rows_v, sem).wait()
        pltpu.sync_copy(rows_v, out_hbm.at[pl.ds(off, _CH)])
        return carry

    lax.fori_loop(0, nch, body, 0)


def _gather(idx_flat, table):
    mesh = plsc.VectorSubcoreMesh(core_axis_name="c", subcore_axis_name="s")
    k = functools.partial(
        pl.kernel,
        mesh=mesh,
        out_type=jax.ShapeDtypeStruct((_B * _K * _N, _C), jnp.float32),
        scratch_types=[
            pltpu.VMEM((_CH,), jnp.int32),
            pltpu.VMEM((_CH, _C), jnp.float32),
            pltpu.SemaphoreType.DMA,
        ],
        compiler_params=pltpu.CompilerParams(use_tc_tiling_on_sc=False),
    )(_gather_body)
    return k(idx_flat, table)


def _conv_body(xt_ref, g_ref, w2_ref, wc_ref, bb_ref, y_ref, ps_ref, pq_ref):
    xr = xt_ref[0]
    acc = lax.dot_general(
        xr, wc_ref[...], (((1,), (0,)), ((), ())),
        preferred_element_type=jnp.float32, precision=lax.Precision.DEFAULT,
    )
    for k in range(_K):
        acc = acc + lax.dot_general(
            g_ref[0, k], w2_ref[k], (((1,), (0,)), ((), ())),
            preferred_element_type=jnp.float32, precision=lax.Precision.DEFAULT,
        )
    acc = acc + bb_ref[...][None, :]
    i = pl.program_id(1)
    y_ref[0] = acc
    ps_ref[0, i, :] = jnp.sum(acc, axis=0)
    pq_ref[0, i, :] = jnp.sum(acc * acc, axis=0)


def _conv(xt3, g, w2, wc, bvec):
    nb = _N // _RC
    return pl.pallas_call(
        _conv_body,
        grid=(_B, nb),
        in_specs=[
            pl.BlockSpec((1, _RC, _C), lambda b, i: (b, i, 0)),
            pl.BlockSpec((1, _K, _RC, _C), lambda b, i: (b, 0, i, 0)),
            pl.BlockSpec((_K, _C, _O), lambda b, i: (0, 0, 0)),
            pl.BlockSpec((_C, _O), lambda b, i: (0, 0)),
            pl.BlockSpec((_O,), lambda b, i: (0,)),
        ],
        out_specs=[
            pl.BlockSpec((1, _RC, _O), lambda b, i: (b, i, 0)),
            pl.BlockSpec((1, _N // _RC, _O), lambda b, i: (b, 0, 0)),
            pl.BlockSpec((1, _N // _RC, _O), lambda b, i: (b, 0, 0)),
        ],
        out_shape=[
            jax.ShapeDtypeStruct((_B, _N, _O), jnp.float32),
            jax.ShapeDtypeStruct((_B, nb, _O), jnp.float32),
            jax.ShapeDtypeStruct((_B, nb, _O), jnp.float32),
        ],
    )(xt3, g, w2, wc, bvec)


def _norm_body(y_ref, ps_ref, pq_ref, gamma_ref, beta_ref, o_ref):
    cnt = jnp.float32(_B * _N)
    mean = jnp.sum(ps_ref[...], axis=(0, 1)) / cnt
    msq = jnp.sum(pq_ref[...], axis=(0, 1)) / cnt
    var = msq - mean * mean
    scale = gamma_ref[...] * lax.rsqrt(var + 1e-5)
    shift = beta_ref[...] - mean * scale
    z = y_ref[0] * scale[None, :] + shift[None, :]
    o_ref[0] = jnp.where(z >= 0.0, z, 0.2 * z)


def _norm(y, ps, pq, gamma, beta):
    nb = _N // _RC
    return pl.pallas_call(
        _norm_body,
        grid=(_B, nb),
        in_specs=[
            pl.BlockSpec((1, _RC, _O), lambda b, i: (b, i, 0)),
            pl.BlockSpec((_B, nb, _O), lambda b, i: (0, 0, 0)),
            pl.BlockSpec((_B, nb, _O), lambda b, i: (0, 0, 0)),
            pl.BlockSpec((_O,), lambda b, i: (0,)),
            pl.BlockSpec((_O,), lambda b, i: (0,)),
        ],
        out_specs=pl.BlockSpec((1, _RC, _O), lambda b, i: (b, i, 0)),
        out_shape=jax.ShapeDtypeStruct((_B, _N, _O), jnp.float32),
    )(y, ps, pq, gamma, beta)


def kernel(x, W, b, gamma, beta):
    xt = jnp.transpose(x, (0, 2, 1))
    w = W[:, :, 0, :]
    w1 = w[:, :_C, :]
    w2 = w[:, _C:, :]
    wc = jnp.transpose(jnp.sum(w1 - w2, axis=2))
    w2k = jnp.transpose(w2, (2, 1, 0))

    idx = _knn(x, xt)
    table = xt.reshape(_B * _N, _C)
    g = _gather(idx.reshape(-1), table)
    g = g.reshape(_B, _K, _N, _C)
    y, ps, pq = _conv(xt, g, w2k, wc, b)
    out = _norm(y, ps, pq, gamma, beta)
    return jnp.transpose(out, (0, 2, 1))
